# baseline (speedup 1.0000x reference)
import functools

import jax
import jax.numpy as jnp
import numpy as np
from jax import lax
from jax.experimental import pallas as pl
from jax.experimental.pallas import tpu as pltpu
from jax.experimental.pallas import tpu_sc as plsc

TOKENS = 32768
D_MODEL = 768
NUM_CHILDREN = 64
NE = NUM_CHILDREN

BT = 8192
QT = TOKENS // 128
QB = BT // 128
NC, NS, L = 2, 16, 16
NW = NC * NS
TOK_PER_W = TOKENS // NW
BLK = 256
QBLK = BLK // 128
FLAT = BLK * NE
NBLK = TOK_PER_W // BLK
NPAIR = BLK // (2 * L)
UNROLL = 8
NEG_INF = float(np.finfo(np.float32).min)


def _mmT_body(x_ref, w_ref, b_ref, o_ref):
    for qi in range(QB):
        xq = x_ref[pl.ds(qi * 128, 128), :]
        lt = lax.dot_general(
            w_ref[...], xq,
            (((1,), (1,)), ((), ())),
            preferred_element_type=jnp.float32,
        ) + b_ref[...]
        o_ref[qi] = lt


def _matmul_logits_t(x, W, bcol):
    return pl.pallas_call(
        _mmT_body,
        grid=(TOKENS // BT,),
        in_specs=[
            pl.BlockSpec((BT, D_MODEL), lambda i: (i, 0)),
            pl.BlockSpec((NE, D_MODEL), lambda i: (0, 0)),
            pl.BlockSpec((NE, 1), lambda i: (0, 0)),
        ],
        out_specs=pl.BlockSpec((QB, NE, 128), lambda i: (i, 0, 0)),
        out_shape=jax.ShapeDtypeStruct((QT, NE, 128), jnp.float32),
        compiler_params=pltpu.CompilerParams(
            dimension_semantics=("arbitrary",),
        ),
    )(x, W, bcol)


def _merge22(h1, l1, ih1, il1, h2, l2, ih2, il2):
    c1 = h1 >= h2
    hi = jnp.where(c1, h1, h2)
    ihi = jnp.where(c1, ih1, ih2)
    m = jnp.where(c1, h2, h1)
    im = jnp.where(c1, ih2, ih1)
    c2 = l1 >= l2
    lc = jnp.where(c2, l1, l2)
    ilc = jnp.where(c2, il1, il2)
    c3 = m >= lc
    lo = jnp.where(c3, m, lc)
    ilo = jnp.where(c3, im, ilc)
    return hi, lo, ihi, ilo


def _top2_tree8(vals, consts):
    t2 = []
    for p in range(4):
        a, b = vals[2 * p], vals[2 * p + 1]
        c = a >= b
        t2.append((jnp.where(c, a, b), jnp.where(c, b, a),
                   jnp.where(c, consts[2 * p], consts[2 * p + 1]),
                   jnp.where(c, consts[2 * p + 1], consts[2 * p])))
    m1 = _merge22(*t2[0], *t2[1])
    m2 = _merge22(*t2[2], *t2[3])
    return _merge22(*m1, *m2)


def _route_body(lt_hbm, out_hbm,
                kt0, kt1, ob0, ob1, idx1, idx2,
                sin0, sin1, so0, so1):
    wid = lax.axis_index("s") * NC + lax.axis_index("c")
    tok0 = wid * TOK_PER_W
    base = tok0 * NE
    ktb = (kt0, kt1)
    obb = (ob0, ob1)
    sin = (sin0, sin1)
    sout = (so0, so1)

    q00 = tok0 // 128
    in_dmas = [
        pltpu.async_copy(lt_hbm.at[pl.ds(q00, QBLK), :, :], kt0, sin0),
        pltpu.async_copy(lt_hbm.at[pl.ds(q00 + QBLK, QBLK), :, :], kt1, sin1),
    ]

    lanes = lax.iota(jnp.int32, L)
    lanes64 = lanes * NE
    neg = jnp.full((L,), NEG_INF, jnp.float32)
    zero_i = jnp.zeros((L,), jnp.int32)
    zero_f = jnp.zeros((L,), jnp.float32)
    consts = [jnp.full((L,), t, jnp.int32) for t in range(UNROLL)]

    def zloop(i, _):
        for k in range(8):
            ob0[pl.ds(i * 128 + k * L, L)] = zero_f
            ob1[pl.ds(i * 128 + k * L, L)] = zero_f
        return 0
    lax.fori_loop(0, FLAT // 128, zloop, 0)

    out_dmas = [None, None]
    for g in range(NBLK):
        bsel = g % 2
        kt = ktb[bsel]
        ob = obb[bsel]
        in_dmas[bsel].wait()
        if out_dmas[bsel] is not None:
            out_dmas[bsel].wait()

            def rloop(up, _):
                off = bsel * BLK + up * (2 * L)
                for d in range(2):
                    o1 = idx1[pl.ds(off + d * L, L)]
                    o2 = idx2[pl.ds(off + d * L, L)]
                    plsc.store_scatter(ob, [o1], zero_f)
                    plsc.store_scatter(ob, [o2], zero_f)
                return 0
            lax.fori_loop(0, NPAIR, rloop, 0)

        def pair(up, _):
            ua = up * (2 * L)
            qa = ua // 128
            ca = ua % 128

            def step(s, carry):
                ha, la, iha, ila, hb, lb, ihb, ilb = carry
                s8 = s * UNROLL
                va = [kt[qa, s8 + t, pl.ds(ca, L)] for t in range(UNROLL)]
                vb = [kt[qa, s8 + t, pl.ds(ca + L, L)] for t in range(UNROLL)]
                s8v = jnp.full((L,), 1, jnp.int32) * s8
                tha, tla, tia, tila = _top2_tree8(va, consts)
                thb, tlb, tib, tilb = _top2_tree8(vb, consts)
                ha, la, iha, ila = _merge22(
                    ha, la, iha, ila, tha, tla, tia + s8v, tila + s8v)
                hb, lb, ihb, ilb = _merge22(
                    hb, lb, ihb, ilb, thb, tlb, tib + s8v, tilb + s8v)
                return (ha, la, iha, ila, hb, lb, ihb, ilb)

            init = (neg, neg, zero_i, zero_i, neg, neg, zero_i, zero_i)
            (ha, la, iha, ila,
             hb, lb, ihb, ilb) = lax.fori_loop(0, NE // UNROLL, step, init)

            ta = ua * NE + lanes64
            tb = ta + L * NE
            off = bsel * BLK + up * (2 * L)
            for d, (h, l, ih, il, tvec) in enumerate(
                    ((ha, la, iha, ila, ta), (hb, lb, ihb, ilb, tb))):
                ex = jnp.exp(l - h)
                w1 = 1.0 / (1.0 + ex)
                w2 = ex * w1
                s1 = tvec + ih
                s2 = tvec + il
                plsc.store_scatter(ob, [s1], w1)
                plsc.store_scatter(ob, [s2], w2)
                idx1[pl.ds(off + d * L, L)] = s1
                idx2[pl.ds(off + d * L, L)] = s2
            return 0

        lax.fori_loop(0, NPAIR, pair, 0)

        out_dmas[bsel] = pltpu.async_copy(
            ob, out_hbm.at[pl.ds(base + g * FLAT, FLAT)], sout[bsel])
        if g + 2 < NBLK:
            in_dmas[bsel] = pltpu.async_copy(
                lt_hbm.at[pl.ds(q00 + (g + 2) * QBLK, QBLK), :, :],
                kt, sin[bsel])

    out_dmas[0].wait()
    out_dmas[1].wait()


def _route(logits_t):
    mesh = plsc.VectorSubcoreMesh(core_axis_name="c", subcore_axis_name="s")
    return pl.kernel(
        _route_body,
        out_type=jax.ShapeDtypeStruct((TOKENS * NE,), jnp.float32),
        mesh=mesh,
        compiler_params=pltpu.CompilerParams(needs_layout_passes=False),
        scratch_types=[
            pltpu.VMEM((QBLK, NE, 128), jnp.float32),
            pltpu.VMEM((QBLK, NE, 128), jnp.float32),
            pltpu.VMEM((FLAT,), jnp.float32),
            pltpu.VMEM((FLAT,), jnp.float32),
            pltpu.VMEM((2 * BLK,), jnp.int32),
            pltpu.VMEM((2 * BLK,), jnp.int32),
        ] + [pltpu.SemaphoreType.DMA] * 4,
    )(logits_t)


def kernel(payload_tensor, W, b):
    bcol = b.reshape(NE, 1)
    lt = _matmul_logits_t(payload_tensor, W, bcol)
    out_flat = _route(lt)
    return out_flat.reshape(TOKENS, NE)


# speedup vs baseline: 1.0298x; 1.0298x over previous
import functools

import jax
import jax.numpy as jnp
import numpy as np
from jax import lax
from jax.experimental import pallas as pl
from jax.experimental.pallas import tpu as pltpu
from jax.experimental.pallas import tpu_sc as plsc

TOKENS = 32768
D_MODEL = 768
NUM_CHILDREN = 64
NE = NUM_CHILDREN

BT = 4096
QT = TOKENS // 128
QB = BT // 128
NC, NS, L = 2, 16, 16
NW = NC * NS
TOK_PER_W = TOKENS // NW
BLK = 256
QBLK = BLK // 128
FLAT = BLK * NE
NBLK = TOK_PER_W // BLK
NPAIR = BLK // (2 * L)
UNROLL = 8
NEG_INF = float(np.finfo(np.float32).min)


def _mmT_body(x_ref, w_ref, b_ref, o_ref):
    for qi in range(QB):
        xq = x_ref[pl.ds(qi * 128, 128), :]
        lt = lax.dot_general(
            w_ref[...], xq,
            (((1,), (1,)), ((), ())),
            preferred_element_type=jnp.float32,
        ) + b_ref[...]
        o_ref[qi] = lt


def _matmul_logits_t(x, W, bcol):
    return pl.pallas_call(
        _mmT_body,
        grid=(TOKENS // BT,),
        in_specs=[
            pl.BlockSpec((BT, D_MODEL), lambda i: (i, 0)),
            pl.BlockSpec((NE, D_MODEL), lambda i: (0, 0)),
            pl.BlockSpec((NE, 1), lambda i: (0, 0)),
        ],
        out_specs=pl.BlockSpec((QB, NE, 128), lambda i: (i, 0, 0)),
        out_shape=jax.ShapeDtypeStruct((QT, NE, 128), jnp.float32),
        compiler_params=pltpu.CompilerParams(
            dimension_semantics=("arbitrary",),
        ),
    )(x, W, bcol)


def _merge22(h1, l1, ih1, il1, h2, l2, ih2, il2):
    c1 = h1 >= h2
    hi = jnp.where(c1, h1, h2)
    ihi = jnp.where(c1, ih1, ih2)
    m = jnp.where(c1, h2, h1)
    im = jnp.where(c1, ih2, ih1)
    c2 = l1 >= l2
    lc = jnp.where(c2, l1, l2)
    ilc = jnp.where(c2, il1, il2)
    c3 = m >= lc
    lo = jnp.where(c3, m, lc)
    ilo = jnp.where(c3, im, ilc)
    return hi, lo, ihi, ilo


def _top2_tree8(vals, consts):
    t2 = []
    for p in range(4):
        a, b = vals[2 * p], vals[2 * p + 1]
        c = a >= b
        t2.append((jnp.where(c, a, b), jnp.where(c, b, a),
                   jnp.where(c, consts[2 * p], consts[2 * p + 1]),
                   jnp.where(c, consts[2 * p + 1], consts[2 * p])))
    m1 = _merge22(*t2[0], *t2[1])
    m2 = _merge22(*t2[2], *t2[3])
    return _merge22(*m1, *m2)


def _route_body(lt_hbm, out_hbm,
                kt0, kt1, ob0, ob1, idx1, idx2,
                sin0, sin1, so0, so1):
    wid = lax.axis_index("s") * NC + lax.axis_index("c")
    tok0 = wid * TOK_PER_W
    base = tok0 * NE
    ktb = (kt0, kt1)
    obb = (ob0, ob1)
    sin = (sin0, sin1)
    sout = (so0, so1)

    q00 = tok0 // 128
    in_dmas = [
        pltpu.async_copy(lt_hbm.at[pl.ds(q00, QBLK), :, :], kt0, sin0),
        pltpu.async_copy(lt_hbm.at[pl.ds(q00 + QBLK, QBLK), :, :], kt1, sin1),
    ]

    lanes = lax.iota(jnp.int32, L)
    lanes64 = lanes * NE
    neg = jnp.full((L,), NEG_INF, jnp.float32)
    zero_i = jnp.zeros((L,), jnp.int32)
    zero_f = jnp.zeros((L,), jnp.float32)
    consts = [jnp.full((L,), t, jnp.int32) for t in range(UNROLL)]

    def zloop(i, _):
        for k in range(8):
            ob0[pl.ds(i * 128 + k * L, L)] = zero_f
            ob1[pl.ds(i * 128 + k * L, L)] = zero_f
        return 0
    lax.fori_loop(0, FLAT // 128, zloop, 0)

    out_dmas = [None, None]
    for g in range(NBLK):
        bsel = g % 2
        kt = ktb[bsel]
        ob = obb[bsel]
        in_dmas[bsel].wait()
        if out_dmas[bsel] is not None:
            out_dmas[bsel].wait()

            def rloop(up, _):
                off = bsel * BLK + up * (2 * L)
                for d in range(2):
                    o1 = idx1[pl.ds(off + d * L, L)]
                    o2 = idx2[pl.ds(off + d * L, L)]
                    plsc.store_scatter(ob, [o1], zero_f)
                    plsc.store_scatter(ob, [o2], zero_f)
                return 0
            lax.fori_loop(0, NPAIR, rloop, 0)

        def pair(up, _):
            ua = up * (2 * L)
            qa = ua // 128
            ca = ua % 128

            def step(s, carry):
                ha, la, iha, ila, hb, lb, ihb, ilb = carry
                s8 = s * UNROLL
                va = [kt[qa, s8 + t, pl.ds(ca, L)] for t in range(UNROLL)]
                vb = [kt[qa, s8 + t, pl.ds(ca + L, L)] for t in range(UNROLL)]
                s8v = jnp.full((L,), 1, jnp.int32) * s8
                tha, tla, tia, tila = _top2_tree8(va, consts)
                thb, tlb, tib, tilb = _top2_tree8(vb, consts)
                ha, la, iha, ila = _merge22(
                    ha, la, iha, ila, tha, tla, tia + s8v, tila + s8v)
                hb, lb, ihb, ilb = _merge22(
                    hb, lb, ihb, ilb, thb, tlb, tib + s8v, tilb + s8v)
                return (ha, la, iha, ila, hb, lb, ihb, ilb)

            init = (neg, neg, zero_i, zero_i, neg, neg, zero_i, zero_i)
            (ha, la, iha, ila,
             hb, lb, ihb, ilb) = lax.fori_loop(0, NE // UNROLL, step, init)

            ta = ua * NE + lanes64
            tb = ta + L * NE
            off = bsel * BLK + up * (2 * L)
            for d, (h, l, ih, il, tvec) in enumerate(
                    ((ha, la, iha, ila, ta), (hb, lb, ihb, ilb, tb))):
                ex = jnp.exp(l - h)
                w1 = 1.0 / (1.0 + ex)
                w2 = ex * w1
                s1 = tvec + ih
                s2 = tvec + il
                plsc.store_scatter(ob, [s1], w1)
                plsc.store_scatter(ob, [s2], w2)
                idx1[pl.ds(off + d * L, L)] = s1
                idx2[pl.ds(off + d * L, L)] = s2
            return 0

        lax.fori_loop(0, NPAIR, pair, 0)

        out_dmas[bsel] = pltpu.async_copy(
            ob, out_hbm.at[pl.ds(base + g * FLAT, FLAT)], sout[bsel])
        if g + 2 < NBLK:
            in_dmas[bsel] = pltpu.async_copy(
                lt_hbm.at[pl.ds(q00 + (g + 2) * QBLK, QBLK), :, :],
                kt, sin[bsel])

    out_dmas[0].wait()
    out_dmas[1].wait()


def _route(logits_t):
    mesh = plsc.VectorSubcoreMesh(core_axis_name="c", subcore_axis_name="s")
    return pl.kernel(
        _route_body,
        out_type=jax.ShapeDtypeStruct((TOKENS * NE,), jnp.float32),
        mesh=mesh,
        compiler_params=pltpu.CompilerParams(needs_layout_passes=False),
        scratch_types=[
            pltpu.VMEM((QBLK, NE, 128), jnp.float32),
            pltpu.VMEM((QBLK, NE, 128), jnp.float32),
            pltpu.VMEM((FLAT,), jnp.float32),
            pltpu.VMEM((FLAT,), jnp.float32),
            pltpu.VMEM((2 * BLK,), jnp.int32),
            pltpu.VMEM((2 * BLK,), jnp.int32),
        ] + [pltpu.SemaphoreType.DMA] * 4,
    )(logits_t)


def kernel(payload_tensor, W, b):
    bcol = b.reshape(NE, 1)
    lt = _matmul_logits_t(payload_tensor, W, bcol)
    out_flat = _route(lt)
    return out_flat.reshape(TOKENS, NE)
